# baseline (speedup 1.0000x reference)
import functools

import jax
import jax.numpy as jnp
from jax import lax
from jax.experimental import pallas as pl
from jax.experimental.pallas import tpu as pltpu
from jax.experimental.pallas import tpu_sc as plsc

_info = plsc.get_sparse_core_info()
_NC, _NS = _info.num_cores, _info.num_subcores
_NW = _NC * _NS

_HPAD = 56
_NBUF = 4


def _make_gather(batch, hist, dim):
  b_per_w = batch // _NW
  assert b_per_w % _NBUF == 0
  mesh = plsc.VectorSubcoreMesh(core_axis_name="c", subcore_axis_name="s")

  @functools.partial(
      pl.kernel,
      mesh=mesh,
      out_type=jax.ShapeDtypeStruct((batch, hist, dim), jnp.float32),
      scratch_types=[
          pltpu.VMEM((b_per_w, _HPAD), jnp.int32),
          pltpu.VMEM((_NBUF, _HPAD, 2 * dim), jnp.float32),
      ]
      + [pltpu.SemaphoreType.DMA] * (2 * _NBUF),
      compiler_params=pltpu.CompilerParams(use_tc_tiling_on_sc=False),
  )
  def gather_kernel(idx_hbm, w3_hbm, out_hbm, idx_v, wide_v, *sems):
    gsem = sems[:_NBUF]
    ssem = sems[_NBUF:]
    wid = lax.axis_index("s") * _NC + lax.axis_index("c")
    base = wid * b_per_w
    pltpu.sync_copy(idx_hbm.at[wid], idx_v)

    for b in range(_NBUF):
      pltpu.async_copy(w3_hbm.at[idx_v.at[b]], wide_v.at[b], gsem[b])

    def outer(g, carry):
      for b in range(_NBUF):
        t = g * _NBUF + b
        pltpu.make_async_copy(
            w3_hbm.at[idx_v.at[t]], wide_v.at[b], gsem[b]
        ).wait()
        src = wide_v.at[b, pl.ds(0, hist), pl.ds(0, dim)]
        dst = out_hbm.at[base + t]
        pltpu.async_copy(src, dst, ssem[b])
        pltpu.make_async_copy(src, dst, ssem[b]).wait()

        @pl.when(t + _NBUF < b_per_w)
        def _():
          pltpu.async_copy(
              w3_hbm.at[idx_v.at[t + _NBUF]], wide_v.at[b], gsem[b]
          )

      return carry

    lax.fori_loop(0, b_per_w // _NBUF, outer, 0)

  return gather_kernel


def kernel(input, weight):
  batch, hist = input.shape
  vocab, dim = weight.shape
  w3 = jnp.pad(weight, ((0, 0), (0, dim)))
  idx = jnp.pad(input.astype(jnp.int32), ((0, 0), (0, _HPAD - hist)))
  idx = idx.reshape(_NW, batch // _NW, _HPAD)
  return _make_gather(batch, hist, dim)(idx, w3)


# speedup vs baseline: 4.2050x; 4.2050x over previous
import functools

import jax
import jax.numpy as jnp
from jax import lax
from jax.experimental import pallas as pl
from jax.experimental.pallas import tpu as pltpu
from jax.experimental.pallas import tpu_sc as plsc

_info = plsc.get_sparse_core_info()
_NC, _NS = _info.num_cores, _info.num_subcores
_NW = _NC * _NS

_CHUNK = 128
_NBUF = 4


def _make_gather(vocab, dim, batch):
  assert batch % (_NW * _CHUNK) == 0
  b_per_w = batch // _NW
  n_chunks = b_per_w // _CHUNK
  assert n_chunks % _NBUF == 0
  n_outer = n_chunks // _NBUF
  mesh = plsc.VectorSubcoreMesh(core_axis_name="c", subcore_axis_name="s")

  @functools.partial(
      pl.kernel,
      mesh=mesh,
      out_type=jax.ShapeDtypeStruct((batch, dim), jnp.float32),
      scratch_types=[
          pltpu.VMEM((n_chunks, _CHUNK), jnp.int32),
          pltpu.VMEM((_NBUF, _CHUNK, 2 * dim), jnp.float32),
      ]
      + [pltpu.SemaphoreType.DMA] * (2 * _NBUF),
      compiler_params=pltpu.CompilerParams(use_tc_tiling_on_sc=False),
  )
  def gather_kernel(idx_hbm, w3_hbm, out_hbm, idx_v, wide_v, *sems):
    gsem = sems[:_NBUF]
    ssem = sems[_NBUF:]
    wid = lax.axis_index("s") * _NC + lax.axis_index("c")
    base = wid * b_per_w
    pltpu.sync_copy(idx_hbm.at[wid], idx_v)

    for b in range(_NBUF):
      pltpu.async_copy(w3_hbm.at[idx_v.at[b]], wide_v.at[b], gsem[b])

    def outer(g, carry):
      for b in range(_NBUF):
        t = g * _NBUF + b
        pltpu.make_async_copy(
            w3_hbm.at[idx_v.at[t]], wide_v.at[b], gsem[b]
        ).wait()
        src = wide_v.at[b, pl.ds(0, _CHUNK), pl.ds(0, dim)]
        dst = out_hbm.at[pl.ds(base + t * _CHUNK, _CHUNK)]
        pltpu.async_copy(src, dst, ssem[b])
        pltpu.make_async_copy(src, dst, ssem[b]).wait()

        @pl.when(t + _NBUF < n_chunks)
        def _():
          pltpu.async_copy(
              w3_hbm.at[idx_v.at[t + _NBUF]], wide_v.at[b], gsem[b]
          )

      return carry

    lax.fori_loop(0, n_outer, outer, 0)

  return gather_kernel


def kernel(input, weight):
  batch, hist = input.shape
  vocab, dim = weight.shape
  flat = batch * hist
  w3 = jnp.pad(weight, ((0, 0), (0, dim)))
  idx = input.reshape(_NW, flat // (_NW * _CHUNK), _CHUNK).astype(jnp.int32)
  out = _make_gather(vocab, dim, flat)(idx, w3)
  return out.reshape(batch, hist, dim)


# speedup vs baseline: 4.2055x; 1.0001x over previous
import functools

import jax
import jax.numpy as jnp
from jax import lax
from jax.experimental import pallas as pl
from jax.experimental.pallas import tpu as pltpu
from jax.experimental.pallas import tpu_sc as plsc

_info = plsc.get_sparse_core_info()
_NC, _NS = _info.num_cores, _info.num_subcores
_NW = _NC * _NS

_CHUNK = 128
_NBUF = 4


def _make_gather(vocab, dim, batch):
  assert batch % (_NW * _CHUNK) == 0
  b_per_w = batch // _NW
  n_chunks = b_per_w // _CHUNK
  assert n_chunks % _NBUF == 0
  n_outer = n_chunks // _NBUF
  mesh = plsc.VectorSubcoreMesh(core_axis_name="c", subcore_axis_name="s")

  @functools.partial(
      pl.kernel,
      mesh=mesh,
      out_type=jax.ShapeDtypeStruct((batch, dim), jnp.float32),
      scratch_types=[
          pltpu.VMEM((n_chunks, _CHUNK), jnp.int32),
          pltpu.VMEM((_NBUF, _CHUNK, 2 * dim), jnp.float32),
      ]
      + [pltpu.SemaphoreType.DMA] * (2 * _NBUF),
      compiler_params=pltpu.CompilerParams(use_tc_tiling_on_sc=False),
  )
  def gather_kernel(idx_hbm, w3_hbm, out_hbm, idx_v, wide_v, *sems):
    gsem = sems[:_NBUF]
    ssem = sems[_NBUF:]
    wid = lax.axis_index("s") * _NC + lax.axis_index("c")
    base = wid * b_per_w
    pltpu.sync_copy(idx_hbm.at[wid], idx_v)

    for b in range(_NBUF):
      pltpu.async_copy(w3_hbm.at[idx_v.at[b]], wide_v.at[b], gsem[b])

    def _store_desc(b, row0):
      src = wide_v.at[b, pl.ds(0, _CHUNK), pl.ds(0, dim)]
      dst = out_hbm.at[pl.ds(row0, _CHUNK)]
      return src, dst

    def outer(g, carry):
      for b in range(_NBUF):
        t = g * _NBUF + b
        pb = (b - 1) % _NBUF
        nt = t - 1 + _NBUF
        cond = (t > 0) & (nt < n_chunks) if b == 0 else (nt < n_chunks)

        @pl.when(cond)
        def _():
          s, d = _store_desc(pb, base)
          pltpu.make_async_copy(s, d, ssem[pb]).wait()
          pltpu.async_copy(w3_hbm.at[idx_v.at[nt]], wide_v.at[pb], gsem[pb])

        pltpu.make_async_copy(
            w3_hbm.at[idx_v.at[t]], wide_v.at[b], gsem[b]
        ).wait()
        s, d = _store_desc(b, base + t * _CHUNK)
        pltpu.async_copy(s, d, ssem[b])

      return carry

    lax.fori_loop(0, n_outer, outer, 0)

    for b in range(_NBUF):
      s, d = _store_desc(b, base)
      pltpu.make_async_copy(s, d, ssem[b]).wait()

  return gather_kernel


def kernel(input, weight):
  batch, hist = input.shape
  vocab, dim = weight.shape
  flat = batch * hist
  w3 = jnp.pad(weight, ((0, 0), (0, dim)))
  idx = input.reshape(_NW, flat // (_NW * _CHUNK), _CHUNK).astype(jnp.int32)
  out = _make_gather(vocab, dim, flat)(idx, w3)
  return out.reshape(batch, hist, dim)


# speedup vs baseline: 4.2397x; 1.0081x over previous
import functools

import jax
import jax.numpy as jnp
from jax import lax
from jax.experimental import pallas as pl
from jax.experimental.pallas import tpu as pltpu
from jax.experimental.pallas import tpu_sc as plsc

_info = plsc.get_sparse_core_info()
_NC, _NS = _info.num_cores, _info.num_subcores
_NW = _NC * _NS

_CHUNK = 128
_NBUF = 4


def _make_gather(vocab, dim, batch):
  assert batch % (_NW * _CHUNK) == 0
  b_per_w = batch // _NW
  n_chunks = b_per_w // _CHUNK
  assert n_chunks % _NBUF == 0
  n_outer = n_chunks // _NBUF
  mesh = plsc.VectorSubcoreMesh(core_axis_name="c", subcore_axis_name="s")

  @functools.partial(
      pl.kernel,
      mesh=mesh,
      out_type=jax.ShapeDtypeStruct((batch, dim), jnp.float32),
      scratch_types=[
          pltpu.VMEM((n_chunks, _CHUNK), jnp.int32),
          pltpu.VMEM((_NBUF, _CHUNK, dim), jnp.float32),
      ]
      + [pltpu.SemaphoreType.DMA] * (2 * _NBUF),
      compiler_params=pltpu.CompilerParams(use_tc_tiling_on_sc=False),
  )
  def gather_kernel(idx_hbm, table_hbm, out_hbm, idx_v, rows_v, *sems):
    gsem = sems[:_NBUF]
    ssem = sems[_NBUF:]
    wid = lax.axis_index("s") * _NC + lax.axis_index("c")
    base = wid * b_per_w
    pltpu.sync_copy(idx_hbm.at[wid], idx_v)

    for b in range(_NBUF):
      pltpu.async_copy(table_hbm.at[idx_v.at[b]], rows_v.at[b], gsem[b])

    def outer(g, carry):
      for b in range(_NBUF):
        j = g * _NBUF + b
        pltpu.make_async_copy(
            table_hbm.at[idx_v.at[j]], rows_v.at[b], gsem[b]
        ).wait()
        pltpu.async_copy(
            rows_v.at[b], out_hbm.at[pl.ds(base + j * _CHUNK, _CHUNK)], ssem[b]
        )
        pltpu.make_async_copy(
            rows_v.at[b], out_hbm.at[pl.ds(base + j * _CHUNK, _CHUNK)], ssem[b]
        ).wait()

        @pl.when(g < n_outer - 1)
        def _():
          pltpu.async_copy(
              table_hbm.at[idx_v.at[j + _NBUF]], rows_v.at[b], gsem[b]
          )

      return carry

    lax.fori_loop(0, n_outer, outer, 0)

  return gather_kernel


def kernel(input, weight):
  batch, hist = input.shape
  vocab, dim = weight.shape
  flat = batch * hist
  idx = input.reshape(_NW, flat // (_NW * _CHUNK), _CHUNK).astype(jnp.int32)
  out = _make_gather(vocab, dim, flat)(idx, weight)
  return out.reshape(batch, hist, dim)
